# baseline (device time: 19873 ns/iter reference)
import jax
import jax.numpy as jnp
from jax import lax
from jax.experimental import pallas as pl
from jax.experimental.pallas import tpu as pltpu

NC = 16


def kernel(x, pi):
    x2 = x[0]
    m, n = x2.shape
    ch = m // NC

    def body(x_ref, pi_ref, out_ref, q_out, q_in, s_out, s_in,
             send_q, recv_q, send_s, recv_s):
        my_x = lax.axis_index("x")
        my_y = lax.axis_index("y")
        dst_y = pi_ref[my_y]

        @pl.when(dst_y != my_y)
        def _swap():
            amax = jnp.max(jnp.abs(x_ref[0:8, :]))
            s = jnp.maximum(amax, 1e-20) * (1.2 / 127.0)
            rcp = 1.0 / s
            s_out[...] = jnp.full((8, 128), s, jnp.float32)

            barrier = pltpu.get_barrier_semaphore()
            pl.semaphore_signal(
                barrier, inc=1,
                device_id=(my_x, dst_y), device_id_type=pl.DeviceIdType.MESH,
            )
            pl.semaphore_wait(barrier, 1)

            rdma_s = pltpu.make_async_remote_copy(
                src_ref=s_out, dst_ref=s_in,
                send_sem=send_s, recv_sem=recv_s,
                device_id=(my_x, dst_y),
                device_id_type=pl.DeviceIdType.MESH,
            )
            rdma_s.start()

            rdma_q = []
            for c in range(NC):
                r = c * ch
                q_out[r:r + ch, :] = jnp.clip(
                    jnp.round(x_ref[r:r + ch, :] * rcp), -127.0, 127.0
                ).astype(jnp.int8)
                d = pltpu.make_async_remote_copy(
                    src_ref=q_out.at[pl.ds(r, ch), :],
                    dst_ref=q_in.at[pl.ds(r, ch), :],
                    send_sem=send_q.at[c],
                    recv_sem=recv_q.at[c],
                    device_id=(my_x, dst_y),
                    device_id_type=pl.DeviceIdType.MESH,
                )
                d.start()
                rdma_q.append(d)

            rdma_s.wait_recv()
            s_peer = s_in[0, 0].astype(jnp.bfloat16)
            for c in range(NC):
                rdma_q[c].wait_recv()
                r = c * ch
                out_ref[r:r + ch, :] = (
                    q_in[r:r + ch, :].astype(jnp.bfloat16) * s_peer
                )

            rdma_s.wait_send()
            for c in range(NC):
                rdma_q[c].wait_send()

        @pl.when(dst_y == my_y)
        def _identity():
            out_ref[...] = x_ref[...].astype(jnp.bfloat16)

    out = pl.pallas_call(
        body,
        out_shape=jax.ShapeDtypeStruct((m, n), jnp.bfloat16),
        in_specs=[
            pl.BlockSpec(memory_space=pltpu.VMEM),
            pl.BlockSpec(memory_space=pltpu.SMEM),
        ],
        out_specs=pl.BlockSpec(memory_space=pltpu.VMEM),
        scratch_shapes=[
            pltpu.VMEM((m, n), jnp.int8),
            pltpu.VMEM((m, n), jnp.int8),
            pltpu.VMEM((8, 128), jnp.float32),
            pltpu.VMEM((8, 128), jnp.float32),
            pltpu.SemaphoreType.DMA((NC,)),
            pltpu.SemaphoreType.DMA((NC,)),
            pltpu.SemaphoreType.DMA,
            pltpu.SemaphoreType.DMA,
        ],
        compiler_params=pltpu.CompilerParams(collective_id=0),
    )(x2, pi)
    return out[None]


# device time: 17177 ns/iter; 1.1570x vs baseline; 1.1570x over previous
import jax
import jax.numpy as jnp
from jax import lax
from jax.experimental import pallas as pl
from jax.experimental.pallas import tpu as pltpu

NC = 8


def kernel(x, pi):
    _, m, n = x.shape
    half = m // 2
    ch = half // NC

    def body(x_ref, pi_ref, out_ref, q_out, q_in, s_out, s_in,
             send1q, recv1q, send2q, recv2q,
             send1s, recv1s, send2s, recv2s):
        my_x = lax.axis_index("x")
        my_y = lax.axis_index("y")
        dst_y = pi_ref[my_y]

        @pl.when(dst_y != my_y)
        def _swap():
            for mx in (0, 1):
                @pl.when(my_x == mx)
                def _(mx=mx):
                    base = mx * half
                    pbase = (1 - mx) * half

                    amax = jnp.max(jnp.abs(x_ref[base:base + 8, :]))
                    s = jnp.maximum(amax, 1e-20) * (1.2 / 127.0)
                    rcp = 1.0 / s
                    s_out[...] = jnp.full((8, 128), s, jnp.float32)

                    barrier = pltpu.get_barrier_semaphore()
                    for nbr in ((mx, dst_y), (1 - mx, my_y)):
                        pl.semaphore_signal(
                            barrier, inc=1,
                            device_id=nbr,
                            device_id_type=pl.DeviceIdType.MESH,
                        )
                    pl.semaphore_wait(barrier, 2)

                    rdma1s = pltpu.make_async_remote_copy(
                        src_ref=s_out,
                        dst_ref=s_in.at[0],
                        send_sem=send1s, recv_sem=recv1s,
                        device_id=(mx, dst_y),
                        device_id_type=pl.DeviceIdType.MESH,
                    )
                    rdma1s.start()

                    rdma1 = []
                    for c in range(NC):
                        r = base + c * ch
                        q_out[r:r + ch, :] = jnp.clip(
                            jnp.round(x_ref[r:r + ch, :] * rcp),
                            -127.0, 127.0,
                        ).astype(jnp.int8)
                        d = pltpu.make_async_remote_copy(
                            src_ref=q_out.at[pl.ds(r, ch), :],
                            dst_ref=q_in.at[pl.ds(r, ch), :],
                            send_sem=send1q.at[c],
                            recv_sem=recv1q.at[c],
                            device_id=(mx, dst_y),
                            device_id_type=pl.DeviceIdType.MESH,
                        )
                        d.start()
                        rdma1.append(d)

                    rdma1s.wait_recv()
                    rdma2s = pltpu.make_async_remote_copy(
                        src_ref=s_in.at[0],
                        dst_ref=s_in.at[1],
                        send_sem=send2s, recv_sem=recv2s,
                        device_id=(1 - mx, my_y),
                        device_id_type=pl.DeviceIdType.MESH,
                    )
                    rdma2s.start()
                    s_y = s_in[0, 0, 0].astype(jnp.bfloat16)

                    rdma2 = []
                    for c in range(NC):
                        rdma1[c].wait_recv()
                        r = base + c * ch
                        d = pltpu.make_async_remote_copy(
                            src_ref=q_in.at[pl.ds(r, ch), :],
                            dst_ref=q_in.at[pl.ds(r, ch), :],
                            send_sem=send2q.at[c],
                            recv_sem=recv2q.at[c],
                            device_id=(1 - mx, my_y),
                            device_id_type=pl.DeviceIdType.MESH,
                        )
                        d.start()
                        rdma2.append(d)
                        out_ref[r:r + ch, :] = (
                            q_in[r:r + ch, :].astype(jnp.bfloat16) * s_y
                        )

                    rdma2s.wait_recv()
                    s_x = s_in[1, 0, 0].astype(jnp.bfloat16)
                    for c in range(NC):
                        rdma2[c].wait_recv()
                        r = pbase + c * ch
                        out_ref[r:r + ch, :] = (
                            q_in[r:r + ch, :].astype(jnp.bfloat16) * s_x
                        )

                    rdma1s.wait_send()
                    rdma2s.wait_send()
                    for c in range(NC):
                        rdma1[c].wait_send()
                        rdma2[c].wait_send()

        @pl.when(dst_y == my_y)
        def _identity():
            out_ref[...] = x_ref[...].astype(jnp.bfloat16)

    return pl.pallas_call(
        body,
        out_shape=jax.ShapeDtypeStruct((1, m, n), jnp.bfloat16),
        in_specs=[
            pl.BlockSpec((None, m, n), lambda: (0, 0, 0)),
            pl.BlockSpec(memory_space=pltpu.SMEM),
        ],
        out_specs=pl.BlockSpec((None, m, n), lambda: (0, 0, 0)),
        scratch_shapes=[
            pltpu.VMEM((m, n), jnp.int8),
            pltpu.VMEM((m, n), jnp.int8),
            pltpu.VMEM((8, 128), jnp.float32),
            pltpu.VMEM((2, 8, 128), jnp.float32),
            pltpu.SemaphoreType.DMA((NC,)),
            pltpu.SemaphoreType.DMA((NC,)),
            pltpu.SemaphoreType.DMA((NC,)),
            pltpu.SemaphoreType.DMA((NC,)),
            pltpu.SemaphoreType.DMA,
            pltpu.SemaphoreType.DMA,
            pltpu.SemaphoreType.DMA,
            pltpu.SemaphoreType.DMA,
        ],
        compiler_params=pltpu.CompilerParams(collective_id=0),
    )(x, pi)
